# Initial kernel scaffold; baseline (speedup 1.0000x reference)
import jax
import jax.numpy as jnp
from jax.experimental import pallas as pl


def kernel(x, edge_index, edge_attr, batch, params):
    raise NotImplementedError("write your pallas kernel here")


# speedup vs baseline: 14.3895x; 14.3895x over previous
import functools

import jax
import jax.numpy as jnp
from jax import lax
from jax.experimental import pallas as pl
from jax.experimental.pallas import tpu as pltpu
from jax.experimental.pallas import tpu_sc as plsc

N = 10000
E = 320000
D = 128
H = 8
C = 16
ED = 16
NL = 4
B = 256

NPAD = 10240
AR = 10240
EPAD = 323584
EPT = EPAD // 32
KE = 64
NCHUNK = EPT // KE
RB = 2560
EB = 512
F32 = jnp.float32


def _kpre_body(src_ref, dst_ref, ea_ref, we_ref, ee4_ref, easum_ref, cnt_ref,
               dste_ref):
    i = pl.program_id(0)
    sb = src_ref[0, :, :]
    db = dst_ref[0, :, :]
    keepf = (sb != db).astype(F32)
    dste_ref[0, :, :] = jnp.where(sb == db, N, db)
    eab = ea_ref[...]
    for l in range(NL):
        ee4_ref[l, :, :] = jnp.dot(eab, we_ref[l], preferred_element_type=F32)

    @pl.when(i == 0)
    def _():
        easum_ref[...] = jnp.zeros_like(easum_ref)
        cnt_ref[...] = jnp.zeros_like(cnt_ref)

    easum_ref[...] += jnp.broadcast_to(
        jnp.dot(keepf, eab, preferred_element_type=F32), (8, ED))
    cnt_ref[...] += jnp.broadcast_to(jnp.sum(keepf)[None, None], (8, 128))


def _kpre(src3, dst3, eap, We):
    nblk = EPAD // EB
    return pl.pallas_call(
        _kpre_body,
        grid=(nblk,),
        in_specs=[
            pl.BlockSpec((1, 1, EB), lambda i: (i, 0, 0)),
            pl.BlockSpec((1, 1, EB), lambda i: (i, 0, 0)),
            pl.BlockSpec((EB, ED), lambda i: (i, 0)),
            pl.BlockSpec((NL, ED, D), lambda i: (0, 0, 0)),
        ],
        out_specs=[
            pl.BlockSpec((NL, EB, D), lambda i: (0, i, 0)),
            pl.BlockSpec((8, ED), lambda i: (0, 0)),
            pl.BlockSpec((8, 128), lambda i: (0, 0)),
            pl.BlockSpec((1, 1, EB), lambda i: (i, 0, 0)),
        ],
        out_shape=[
            jax.ShapeDtypeStruct((NL, EPAD, D), F32),
            jax.ShapeDtypeStruct((8, ED), F32),
            jax.ShapeDtypeStruct((8, 128), F32),
            jax.ShapeDtypeStruct((EPAD // EB, 1, EB), jnp.int32),
        ],
    )(src3, dst3, eap, We)


def _k0a_body(x_ref, win_ref, bin_ref, easum_ref, cnt_ref, we_ref,
              y_ref, ssum_ref, ssq_ref, eeloop_ref):
    i = pl.program_id(0)
    y = jnp.dot(x_ref[...], win_ref[...], preferred_element_type=F32) + bin_ref[0, :]
    y_ref[...] = y
    rid = i * RB + lax.broadcasted_iota(jnp.int32, (RB, 1), 0)
    m = (rid < N).astype(F32)
    ym = y * m

    @pl.when(i == 0)
    def _():
        ssum_ref[...] = jnp.zeros_like(ssum_ref)
        ssq_ref[...] = jnp.zeros_like(ssq_ref)
        ea_mean = easum_ref[0:1, :] / cnt_ref[0, 0]
        for l in range(NL):
            eeloop_ref[l, :] = jnp.dot(
                ea_mean, we_ref[l], preferred_element_type=F32)[0, :]

    ssum_ref[...] += jnp.broadcast_to(jnp.sum(ym, axis=0)[None, :], (8, D))
    ssq_ref[...] += jnp.broadcast_to(jnp.sum(ym * ym, axis=0)[None, :], (8, D))


def _k0a(xp, W_in, b_in, easum, cnt, We):
    return pl.pallas_call(
        _k0a_body,
        grid=(NPAD // RB,),
        in_specs=[
            pl.BlockSpec((RB, D), lambda i: (i, 0)),
            pl.BlockSpec((D, D), lambda i: (0, 0)),
            pl.BlockSpec((1, D), lambda i: (0, 0)),
            pl.BlockSpec((8, ED), lambda i: (0, 0)),
            pl.BlockSpec((8, 128), lambda i: (0, 0)),
            pl.BlockSpec((NL, ED, D), lambda i: (0, 0, 0)),
        ],
        out_specs=[
            pl.BlockSpec((RB, D), lambda i: (i, 0)),
            pl.BlockSpec((8, D), lambda i: (0, 0)),
            pl.BlockSpec((8, D), lambda i: (0, 0)),
            pl.BlockSpec((NL, D), lambda i: (0, 0)),
        ],
        out_shape=[
            jax.ShapeDtypeStruct((NPAD, D), F32),
            jax.ShapeDtypeStruct((8, D), F32),
            jax.ShapeDtypeStruct((8, D), F32),
            jax.ShapeDtypeStruct((NL, D), F32),
        ],
    )(xp, W_in, b_in, easum, cnt, We)


def _knorm_body(y_ref, ssum_ref, ssq_ref, g_ref, b_ref, res_ref, wl_ref, wr_ref,
                h_ref, xl_ref, xr_ref):
    i = pl.program_id(0)
    mean = ssum_ref[0, :] / N
    var = ssq_ref[0, :] / N - mean * mean
    yn = g_ref[0, :] * (y_ref[...] - mean) * lax.rsqrt(var + 1e-5) + b_ref[0, :]
    e = jnp.where(yn > 0, yn, jnp.exp(jnp.minimum(yn, 0.0)) - 1.0)
    rid = i * RB + lax.broadcasted_iota(jnp.int32, (RB, 1), 0)
    h = jnp.where(rid < N, e + res_ref[...], 0.0)
    h_ref[...] = h
    xl_ref[...] = jnp.dot(h, wl_ref[...], preferred_element_type=F32)
    xr_ref[...] = jnp.dot(h, wr_ref[...], preferred_element_type=F32)


def _knorm(y, ssum, ssq, g, b, res, Wl, Wr):
    return pl.pallas_call(
        _knorm_body,
        grid=(NPAD // RB,),
        in_specs=[
            pl.BlockSpec((RB, D), lambda i: (i, 0)),
            pl.BlockSpec((8, D), lambda i: (0, 0)),
            pl.BlockSpec((8, D), lambda i: (0, 0)),
            pl.BlockSpec((1, D), lambda i: (0, 0)),
            pl.BlockSpec((1, D), lambda i: (0, 0)),
            pl.BlockSpec((RB, D), lambda i: (i, 0)),
            pl.BlockSpec((D, D), lambda i: (0, 0)),
            pl.BlockSpec((D, D), lambda i: (0, 0)),
        ],
        out_specs=[
            pl.BlockSpec((RB, D), lambda i: (i, 0)),
            pl.BlockSpec((RB, D), lambda i: (i, 0)),
            pl.BlockSpec((RB, D), lambda i: (i, 0)),
        ],
        out_shape=[
            jax.ShapeDtypeStruct((NPAD, D), F32),
            jax.ShapeDtypeStruct((NPAD, D), F32),
            jax.ShapeDtypeStruct((NPAD, D), F32),
        ],
    )(y, ssum, ssq, g, b, res, Wl, Wr)


def _selectors():
    r = lax.broadcasted_iota(jnp.int32, (D, H), 0) // C
    c = lax.broadcasted_iota(jnp.int32, (D, H), 1)
    return (r == c).astype(F32)


def _self_loop_w(xl, xr, eeloop, attf, S):
    t = xl + xr + eeloop
    t = jnp.maximum(t, 0.2 * t)
    return jnp.exp(jnp.dot(t * attf, S, preferred_element_type=F32))


def _kcomb_body(accw_ref, accs_ref, xl_ref, xr_ref, eeloop_ref, attf_ref,
                bg_ref, y_ref, ssum_ref, ssq_ref):
    i = pl.program_id(0)
    S = _selectors()
    xl = xl_ref[...]
    wl = _self_loop_w(xl, xr_ref[...], eeloop_ref[0, :], attf_ref[0, :], S)
    wlrep = jnp.dot(wl, S.T, preferred_element_type=F32)
    accd = accw_ref[0] + accw_ref[1] + xl * wlrep
    srep = accs_ref[0] + accs_ref[1] + wlrep
    y = accd / (srep + 1e-16) + bg_ref[0, :]
    rid = i * RB + lax.broadcasted_iota(jnp.int32, (RB, 1), 0)
    y = jnp.where(rid < N, y, 0.0)
    y_ref[...] = y

    @pl.when(i == 0)
    def _():
        ssum_ref[...] = jnp.zeros_like(ssum_ref)
        ssq_ref[...] = jnp.zeros_like(ssq_ref)

    ssum_ref[...] += jnp.broadcast_to(jnp.sum(y, axis=0)[None, :], (8, D))
    ssq_ref[...] += jnp.broadcast_to(jnp.sum(y * y, axis=0)[None, :], (8, D))


def _kcomb(accw, accs, xl, xr, eeloop, attf, bg):
    return pl.pallas_call(
        _kcomb_body,
        grid=(NPAD // RB,),
        in_specs=[
            pl.BlockSpec((2, RB, D), lambda i: (0, i, 0)),
            pl.BlockSpec((2, RB, D), lambda i: (0, i, 0)),
            pl.BlockSpec((RB, D), lambda i: (i, 0)),
            pl.BlockSpec((RB, D), lambda i: (i, 0)),
            pl.BlockSpec((1, D), lambda i: (0, 0)),
            pl.BlockSpec((1, D), lambda i: (0, 0)),
            pl.BlockSpec((1, D), lambda i: (0, 0)),
        ],
        out_specs=[
            pl.BlockSpec((RB, D), lambda i: (i, 0)),
            pl.BlockSpec((8, D), lambda i: (0, 0)),
            pl.BlockSpec((8, D), lambda i: (0, 0)),
        ],
        out_shape=[
            jax.ShapeDtypeStruct((NPAD, D), F32),
            jax.ShapeDtypeStruct((8, D), F32),
            jax.ShapeDtypeStruct((8, D), F32),
        ],
    )(accw, accs, xl, xr, eeloop, attf, bg)


def _bn_rows(x, g, b, n):
    mean = jnp.sum(x, axis=0) / n
    var = jnp.sum(x * x, axis=0) / n - mean * mean
    return g * (x - mean) * lax.rsqrt(var + 1e-5) + b


def _kpool_body(y_ref, ssum_ref, ssq_ref, g3_ref, b3_ref, res_ref, batch_ref,
                G1_ref, g1b_ref, G2_ref, g2b_ref,
                R1_ref, r1b_ref, bn1g_ref, bn1b_ref,
                R2_ref, r2b_ref, bn2g_ref, bn2b_ref,
                R3_ref, r3b_ref, R4_ref, r4b_ref, out_ref):
    mean = ssum_ref[0, :] / N
    var = ssq_ref[0, :] / N - mean * mean
    yn = g3_ref[0, :] * (y_ref[...] - mean) * lax.rsqrt(var + 1e-5) + b3_ref[0, :]
    e = jnp.where(yn > 0, yn, jnp.exp(jnp.minimum(yn, 0.0)) - 1.0)
    rid = lax.broadcasted_iota(jnp.int32, (NPAD, 1), 0)
    h = jnp.where(rid < N, e + res_ref[...], 0.0)

    ga = jnp.dot(h, G1_ref[...], preferred_element_type=F32) + g1b_ref[0, :]
    gate = jnp.dot(jnp.maximum(ga, 0.0), G2_ref[...],
                   preferred_element_type=F32) + g2b_ref[0, 0]

    bid = batch_ref[...]
    O = (bid == lax.broadcasted_iota(jnp.int32, (NPAD, B), 1)).astype(F32)
    mx = jnp.max(jnp.where(O > 0, gate, -jnp.inf), axis=0,
                 keepdims=True)
    mx = jnp.where(jnp.isfinite(mx), mx, 0.0)
    ev = jnp.exp(gate - jnp.dot(O, mx.T, preferred_element_type=F32))
    ev = jnp.where(rid < N, ev, 0.0)
    s = lax.dot_general(O, ev, (((0,), (0,)), ((), ())),
                        preferred_element_type=F32)
    gcoef = ev / (jnp.dot(O, s, preferred_element_type=F32) + 1e-16)
    pooled = lax.dot_general(O, gcoef * h,
                             (((0,), (0,)), ((), ())),
                             preferred_element_type=F32)

    r = jnp.dot(pooled, R1_ref[...], preferred_element_type=F32) + r1b_ref[0, :]
    r = jnp.maximum(_bn_rows(r, bn1g_ref[0, :], bn1b_ref[0, :], B), 0.0)
    r = jnp.dot(r, R2_ref[...], preferred_element_type=F32) + r2b_ref[0, :]
    r = jnp.maximum(_bn_rows(r, bn2g_ref[0, :], bn2b_ref[0, :], B), 0.0)
    r = jnp.maximum(jnp.dot(r, R3_ref[...], preferred_element_type=F32)
                    + r3b_ref[0, :], 0.0)
    out_ref[...] = jnp.dot(r, R4_ref[...], preferred_element_type=F32) + r4b_ref[0, 0]


def _kpool(y, ssum, ssq, g3, b3, res, batch2, p):
    args = (y, ssum, ssq, g3, b3, res, batch2,
            p['G1'], p['g1b'].reshape(1, -1), p['G2'], p['g2b'].reshape(1, 1),
            p['R1'], p['r1b'].reshape(1, -1),
            p['bn1_g'].reshape(1, -1), p['bn1_b'].reshape(1, -1),
            p['R2'], p['r2b'].reshape(1, -1),
            p['bn2_g'].reshape(1, -1), p['bn2_b'].reshape(1, -1),
            p['R3'], p['r3b'].reshape(1, -1), p['R4'], p['r4b'].reshape(1, 1))
    return pl.pallas_call(
        _kpool_body,
        out_shape=jax.ShapeDtypeStruct((B, 1), F32),
    )(*args)


_GDN = lax.GatherDimensionNumbers(
    offset_dims=(), collapsed_slice_dims=(0,), start_index_map=(0,))


def _shuffle(v, idx):
    return lax.gather(v, idx[:, None], dimension_numbers=_GDN, slice_sizes=(1,),
                      mode=lax.GatherScatterMode.PROMISE_IN_BOUNDS)


def _ksc_body(xl, xr, srcd, dstd, ee, attd, outw, outs,
              accw, accs2, srcv, dstv, ddiv, dmv, xlb, xrb, eeb, wsb2, pbuf,
              attv, sem1, sem2, sem3):
    c = lax.axis_index("c")
    s = lax.axis_index("s")
    wid = c * 16 + s

    pltpu.sync_copy(attd, attv)

    def _zrow(j, _):
        for k in range(D // 16):
            xlb[j, pl.ds(16 * k, 16)] = jnp.zeros((16,), F32)
            wsb2[j, pl.ds(16 * k, 16)] = jnp.zeros((16,), F32)
        return _
    lax.fori_loop(0, KE, _zrow, 0)

    r0 = s * 640
    for k in range(10):
        pltpu.sync_copy(xlb, accw.at[pl.ds(r0 + k * KE, KE), :])
    q0 = s * 80
    pltpu.sync_copy(wsb2, accs2.at[pl.ds(q0, KE), :])
    pltpu.sync_copy(wsb2.at[pl.ds(0, 16), :], accs2.at[pl.ds(q0 + KE, 16), :])
    plsc.subcore_barrier()

    lane = lax.iota(jnp.int32, 16)

    def _chunk(g, _):
        off = wid * EPT + g * KE
        pltpu.sync_copy(srcd.at[pl.ds(off, KE)], srcv)
        pltpu.sync_copy(dstd.at[pl.ds(off, KE)], dstv)
        cp1 = pltpu.async_copy(xl.at[srcv], xlb, sem1)
        cp2 = pltpu.async_copy(xr.at[dstv], xrb, sem2)
        cp3 = pltpu.async_copy(ee.at[pl.ds(off, KE), :], eeb, sem3)
        for k in range(KE // 16):
            vd = dstv[pl.ds(16 * k, 16)]
            ddiv[pl.ds(16 * k, 16)] = lax.shift_right_logical(vd, 3)
            dmv[k, pl.ds(0, 16)] = lax.bitwise_and(vd, 7)
        cp1.wait()
        cp2.wait()
        cp3.wait()

        def _edge(j, _2):
            ws = jnp.zeros((16,), F32)
            for h in range(H):
                a = xlb[j, pl.ds(16 * h, 16)]
                t = a + xrb[j, pl.ds(16 * h, 16)] + eeb[j, pl.ds(16 * h, 16)]
                t = jnp.maximum(t, 0.2 * t)
                v = t * attv[pl.ds(16 * h, 16)]
                for sh in (8, 4, 2, 1):
                    v = v + _shuffle(v, jnp.bitwise_xor(lane, sh))
                wv = jnp.exp(v)
                xlb[j, pl.ds(16 * h, 16)] = wv * a
                ws = jnp.where(lane == h, wv, ws)
            grp = dmv[j // 16, pl.ds(0, 16)]
            dm = _shuffle(grp, jnp.broadcast_to(lax.rem(j, 16), (16,)))
            for k in range(8):
                mk = jnp.where(dm == k, jnp.ones((16,), F32),
                               jnp.zeros((16,), F32))
                wsb2[j, pl.ds(16 * k, 16)] = ws * mk
            return _2
        lax.fori_loop(0, KE, _edge, 0)

        pltpu.sync_copy(xlb, accw.at[dstv], add=True)
        pltpu.sync_copy(wsb2, accs2.at[ddiv], add=True)
        return _
    lax.fori_loop(0, NCHUNK, _chunk, 0)

    plsc.subcore_barrier()
    for k in range(10):
        pltpu.sync_copy(accw.at[pl.ds(r0 + k * KE, KE), :], xlb)
        pltpu.sync_copy(xlb, outw.at[pl.ds(c * AR + r0 + k * KE, KE), :])
    def _blk(blk, _4):
        pltpu.sync_copy(accs2.at[pl.ds(q0 + blk * 8, 8), :], pbuf)

        def _rep(r, _3):
            for g in range(8):
                pv = pbuf[r, pl.ds(16 * g, 16)]
                for h in range(H):
                    pm = jnp.where(lane == h, pv, jnp.zeros((16,), F32))
                    for sh in (8, 4, 2, 1):
                        pm = pm + _shuffle(pm, jnp.bitwise_xor(lane, sh))
                    wsb2[r * 8 + g, pl.ds(16 * h, 16)] = pm
            return _3
        lax.fori_loop(0, 8, _rep, 0)
        pltpu.sync_copy(wsb2, outs.at[pl.ds(c * AR + s * 640 + blk * KE,
                                            KE), :])
        return _4
    lax.fori_loop(0, 10, _blk, 0)


def _ksc(xl, xr, srcp, dstp, eel, attf):
    mesh = plsc.VectorSubcoreMesh(core_axis_name="c", subcore_axis_name="s")
    f = functools.partial(
        pl.kernel,
        mesh=mesh,
        out_type=[
            jax.ShapeDtypeStruct((2 * AR, D), F32),
            jax.ShapeDtypeStruct((2 * AR, D), F32),
        ],
        scratch_types=[
            pltpu.VMEM_SHARED((AR, D), F32),
            pltpu.VMEM_SHARED((AR // 8, D), F32),
            pltpu.VMEM((KE,), jnp.int32),
            pltpu.VMEM((KE,), jnp.int32),
            pltpu.VMEM((KE,), jnp.int32),
            pltpu.VMEM((KE // 16, 16), jnp.int32),
            pltpu.VMEM((KE, D), F32),
            pltpu.VMEM((KE, D), F32),
            pltpu.VMEM((KE, D), F32),
            pltpu.VMEM((KE, D), F32),
            pltpu.VMEM((8, D), F32),
            pltpu.VMEM((D,), F32),
            pltpu.SemaphoreType.DMA,
            pltpu.SemaphoreType.DMA,
            pltpu.SemaphoreType.DMA,
        ],
    )(_ksc_body)
    return f(xl, xr, srcp, dstp, eel, attf)


def kernel(x, edge_index, edge_attr, batch, params):
    p = params
    src0 = edge_index[0]
    dst0 = edge_index[1]
    zpad = jnp.zeros((EPAD - E,), jnp.int32)
    srcp = jnp.concatenate([src0, zpad])
    dstp = jnp.concatenate([dst0, zpad])
    eap = jnp.concatenate([edge_attr, jnp.zeros((EPAD - E, ED), F32)])
    xp = jnp.concatenate([x, jnp.zeros((NPAD - N, x.shape[1]), F32)])
    src3 = srcp.reshape(EPAD // EB, 1, EB)
    dst3 = dstp.reshape(EPAD // EB, 1, EB)
    batch2 = jnp.concatenate(
        [batch, jnp.full((NPAD - N,), B, jnp.int32)]).reshape(NPAD, 1)

    ee4, easum, cnt, dste3 = _kpre(src3, dst3, eap, p['We'])
    dste = dste3.reshape(EPAD)
    y0, ssum, ssq, eeloop = _k0a(xp, p['W_in'], p['b_in'].reshape(1, D),
                                 easum, cnt, p['We'])
    zres = jnp.zeros((NPAD, D), F32)
    h, xl, xr = _knorm(y0, ssum, ssq, p['bn_in_g'].reshape(1, D),
                       p['bn_in_b'].reshape(1, D), zres, p['Wl'][0], p['Wr'][0])

    attf = p['att'].reshape(NL, D)
    for l in range(NL):
        outw, outs = _ksc(xl, xr, srcp, dste, ee4[l], attf[l])
        accw = outw.reshape(2, AR, D)
        accs = outs.reshape(2, AR, D)
        y, ssum, ssq = _kcomb(accw, accs, xl, xr,
                              eeloop[l].reshape(1, D), attf[l].reshape(1, D),
                              p['b_gat'][l].reshape(1, D))
        if l < NL - 1:
            h, xl, xr = _knorm(y, ssum, ssq, p['bn_g'][l].reshape(1, D),
                               p['bn_b'][l].reshape(1, D), h,
                               p['Wl'][l + 1], p['Wr'][l + 1])
        else:
            out = _kpool(y, ssum, ssq, p['bn_g'][l].reshape(1, D),
                         p['bn_b'][l].reshape(1, D), h, batch2, p)
    return out
